# baseline (speedup 1.0000x reference)
import functools

import jax
import jax.numpy as jnp
from jax import lax
from jax.experimental import pallas as pl
from jax.experimental.pallas import tpu as pltpu
from jax.experimental.pallas import tpu_sc as plsc

B = 8
N = 8192
G = 512
K = 32

_NW = 32
_IPW = (B * G * K) // _NW
_CH = 128
_NCH = _IPW // _CH


def _fps_kernel(xs_ref, ys_ref, zs_ref, center_ref):
    xs = xs_ref[...]
    ys = ys_ref[...]
    zs = zs_ref[...]
    iota = jax.lax.broadcasted_iota(jnp.int32, (B, N), 1)

    dist0 = jnp.full((B, N), jnp.inf, dtype=jnp.float32)
    last0 = jnp.zeros((B, 1), dtype=jnp.int32)

    def body(g, carry):
        dist, last = carry
        sel = iota == last
        px = jnp.sum(jnp.where(sel, xs, 0.0), axis=1, keepdims=True)
        py = jnp.sum(jnp.where(sel, ys, 0.0), axis=1, keepdims=True)
        pz = jnp.sum(jnp.where(sel, zs, 0.0), axis=1, keepdims=True)
        center_ref[pl.ds(g, 1), :, :] = jnp.concatenate(
            [px, py, pz], axis=1
        ).reshape(1, B, 3)
        dx = xs - px
        dy = ys - py
        dz = zs - pz
        d = dx * dx + dy * dy + dz * dz
        dist = jnp.minimum(dist, d)
        m = jnp.max(dist, axis=1, keepdims=True)
        cand = jnp.where(dist == m, iota, N)
        nxt = jnp.min(cand, axis=1, keepdims=True)
        return dist, nxt

    jax.lax.fori_loop(0, G, body, (dist0, last0))


_ROWS = 256


def _knn_kernel(xs_ref, ys_ref, zs_ref, center_ref, idx_ref):
    xsb = xs_ref[0]
    ysb = ys_ref[0]
    zsb = zs_ref[0]
    cx = center_ref[0, :, 0:1]
    cy = center_ref[0, :, 1:2]
    cz = center_ref[0, :, 2:3]
    dx = cx - xsb
    dy = cy - ysb
    dz = cz - zsb
    d = dx * dx + dy * dy + dz * dz
    iota = jax.lax.broadcasted_iota(jnp.int32, (_ROWS, N), 1)
    base = pl.program_id(0) * N

    for k in range(K):
        m = jnp.min(d, axis=1, keepdims=True)
        j = jnp.min(jnp.where(d == m, iota, N), axis=1, keepdims=True)
        idx_ref[0, :, k : k + 1] = j + base
        d = jnp.where(iota == j, jnp.inf, d)


@functools.partial(
    pl.kernel,
    mesh=plsc.VectorSubcoreMesh(core_axis_name="c", subcore_axis_name="s"),
    compiler_params=pltpu.CompilerParams(use_tc_tiling_on_sc=False),
    out_type=jax.ShapeDtypeStruct((B * G * K // _CH, _CH, 8), jnp.float32),
    scratch_types=[
        pltpu.VMEM((_CH,), jnp.int32),
        pltpu.VMEM((_CH, 8), jnp.float32),
        pltpu.SemaphoreType.DMA,
    ],
)
def _sc_gather(tbl_hbm, idx_hbm, out_hbm, idx_v, rows_v, sem):
    wid = lax.axis_index("s") * 2 + lax.axis_index("c")
    c0 = wid * _NCH

    def chunk(j, carry):
        pltpu.sync_copy(idx_hbm.at[c0 + j], idx_v)
        pltpu.async_copy(tbl_hbm.at[idx_v], rows_v, sem).wait()
        pltpu.sync_copy(rows_v, out_hbm.at[c0 + j])
        return carry

    lax.fori_loop(0, _NCH, chunk, 0)


def _sub_kernel(rows_ref, center_ref, nb_ref):
    nb_ref[...] = rows_ref[:, :, 0:3] - center_ref[...]


def kernel(xyz):
    xs = xyz[:, 0, :]
    ys = xyz[:, 1, :]
    zs = xyz[:, 2, :]

    center_t = pl.pallas_call(
        _fps_kernel,
        out_shape=jax.ShapeDtypeStruct((G, B, 3), jnp.float32),
    )(xs, ys, zs)
    center = jnp.transpose(center_t, (1, 0, 2))

    xs3 = xs[:, None, :]
    ys3 = ys[:, None, :]
    zs3 = zs[:, None, :]

    idx = pl.pallas_call(
        _knn_kernel,
        grid=(B, G // _ROWS),
        in_specs=[
            pl.BlockSpec((1, 1, N), lambda b, gb: (b, 0, 0)),
            pl.BlockSpec((1, 1, N), lambda b, gb: (b, 0, 0)),
            pl.BlockSpec((1, 1, N), lambda b, gb: (b, 0, 0)),
            pl.BlockSpec((1, _ROWS, 3), lambda b, gb: (b, gb, 0)),
        ],
        out_specs=pl.BlockSpec((1, _ROWS, K), lambda b, gb: (b, gb, 0)),
        out_shape=jax.ShapeDtypeStruct((B, G, K), jnp.int32),
    )(xs3, ys3, zs3, center)

    z = jnp.zeros_like(xs)
    tbl = jnp.stack([xs, ys, zs, z, z, z, z, z], axis=-1).reshape(B * N, 8)
    rows4 = _sc_gather(tbl, idx.reshape(B * G * K // _CH, _CH))

    neighborhood = pl.pallas_call(
        _sub_kernel,
        grid=(32,),
        in_specs=[
            pl.BlockSpec((128, K, 8), lambda i: (i, 0, 0)),
            pl.BlockSpec((128, 1, 3), lambda i: (i, 0, 0)),
        ],
        out_specs=pl.BlockSpec((128, K, 3), lambda i: (i, 0, 0)),
        out_shape=jax.ShapeDtypeStruct((B * G, K, 3), jnp.float32),
    )(rows4.reshape(B * G, K, 8), center.reshape(B * G, 1, 3))

    return neighborhood.reshape(B, G, K, 3), center


# speedup vs baseline: 1.0496x; 1.0496x over previous
import functools

import jax
import jax.numpy as jnp
from jax import lax
from jax.experimental import pallas as pl
from jax.experimental.pallas import tpu as pltpu
from jax.experimental.pallas import tpu_sc as plsc

B = 8
N = 8192
G = 512
K = 32

_NW = 32
_IPW = (B * G * K) // _NW
_CH = 128
_NCH = _IPW // _CH


def _fps_kernel(xs_ref, ys_ref, zs_ref, center_ref):
    xs = xs_ref[...]
    ys = ys_ref[...]
    zs = zs_ref[...]
    iota = jax.lax.broadcasted_iota(jnp.int32, (B, N), 1)

    dist0 = jnp.full((B, N), jnp.inf, dtype=jnp.float32)
    last0 = jnp.zeros((B, 1), dtype=jnp.int32)

    def step(g, dist, last):
        sel = iota == last
        px = jnp.sum(jnp.where(sel, xs, 0.0), axis=1, keepdims=True)
        py = jnp.sum(jnp.where(sel, ys, 0.0), axis=1, keepdims=True)
        pz = jnp.sum(jnp.where(sel, zs, 0.0), axis=1, keepdims=True)
        center_ref[pl.ds(g, 1), :, :] = jnp.concatenate(
            [px, py, pz], axis=1
        ).reshape(1, B, 3)
        dx = xs - px
        dy = ys - py
        dz = zs - pz
        d = dx * dx + dy * dy + dz * dz
        dist = jnp.minimum(dist, d)
        m = jnp.max(dist, axis=1, keepdims=True)
        cand = jnp.where(dist == m, iota, N)
        nxt = jnp.min(cand, axis=1, keepdims=True)
        return dist, nxt

    def body(g4, carry):
        dist, last = carry
        g = g4 * 4
        for u in range(4):
            dist, last = step(g + u, dist, last)
        return dist, last

    jax.lax.fori_loop(0, G // 4, body, (dist0, last0))


_ROWS = 128


def _knn_kernel(xs_ref, ys_ref, zs_ref, center_ref, idx_ref):
    xsb = xs_ref[0]
    ysb = ys_ref[0]
    zsb = zs_ref[0]
    cx = center_ref[0, :, 0:1]
    cy = center_ref[0, :, 1:2]
    cz = center_ref[0, :, 2:3]
    dx = cx - xsb
    dy = cy - ysb
    dz = cz - zsb
    d = dx * dx + dy * dy + dz * dz
    iota = jax.lax.broadcasted_iota(jnp.int32, (_ROWS, N), 1)
    base = pl.program_id(0) * N

    for k in range(K):
        m = jnp.min(d, axis=1, keepdims=True)
        j = jnp.min(jnp.where(d == m, iota, N), axis=1, keepdims=True)
        idx_ref[0, :, k : k + 1] = j + base
        d = jnp.where(iota == j, jnp.inf, d)


@functools.partial(
    pl.kernel,
    mesh=plsc.VectorSubcoreMesh(core_axis_name="c", subcore_axis_name="s"),
    compiler_params=pltpu.CompilerParams(use_tc_tiling_on_sc=False),
    out_type=jax.ShapeDtypeStruct((B * G * K // _CH, _CH, 8), jnp.float32),
    scratch_types=[
        pltpu.VMEM((_CH,), jnp.int32),
        pltpu.VMEM((_CH, 8), jnp.float32),
        pltpu.SemaphoreType.DMA,
    ],
)
def _sc_gather(tbl_hbm, idx_hbm, out_hbm, idx_v, rows_v, sem):
    wid = lax.axis_index("s") * 2 + lax.axis_index("c")
    c0 = wid * _NCH

    def chunk(j, carry):
        pltpu.sync_copy(idx_hbm.at[c0 + j], idx_v)
        pltpu.async_copy(tbl_hbm.at[idx_v], rows_v, sem).wait()
        pltpu.sync_copy(rows_v, out_hbm.at[c0 + j])
        return carry

    lax.fori_loop(0, _NCH, chunk, 0)


def _sub_kernel(rows_ref, center_ref, nb_ref):
    nb_ref[...] = rows_ref[:, :, 0:3] - center_ref[...]


def kernel(xyz):
    xs = xyz[:, 0, :]
    ys = xyz[:, 1, :]
    zs = xyz[:, 2, :]

    center_t = pl.pallas_call(
        _fps_kernel,
        out_shape=jax.ShapeDtypeStruct((G, B, 3), jnp.float32),
    )(xs, ys, zs)
    center = jnp.transpose(center_t, (1, 0, 2))

    xs3 = xs[:, None, :]
    ys3 = ys[:, None, :]
    zs3 = zs[:, None, :]

    idx = pl.pallas_call(
        _knn_kernel,
        grid=(B, G // _ROWS),
        in_specs=[
            pl.BlockSpec((1, 1, N), lambda b, gb: (b, 0, 0)),
            pl.BlockSpec((1, 1, N), lambda b, gb: (b, 0, 0)),
            pl.BlockSpec((1, 1, N), lambda b, gb: (b, 0, 0)),
            pl.BlockSpec((1, _ROWS, 3), lambda b, gb: (b, gb, 0)),
        ],
        out_specs=pl.BlockSpec((1, _ROWS, K), lambda b, gb: (b, gb, 0)),
        out_shape=jax.ShapeDtypeStruct((B, G, K), jnp.int32),
    )(xs3, ys3, zs3, center)

    z = jnp.zeros_like(xs)
    tbl = jnp.stack([xs, ys, zs, z, z, z, z, z], axis=-1).reshape(B * N, 8)
    rows4 = _sc_gather(tbl, idx.reshape(B * G * K // _CH, _CH))

    neighborhood = pl.pallas_call(
        _sub_kernel,
        grid=(32,),
        in_specs=[
            pl.BlockSpec((128, K, 8), lambda i: (i, 0, 0)),
            pl.BlockSpec((128, 1, 3), lambda i: (i, 0, 0)),
        ],
        out_specs=pl.BlockSpec((128, K, 3), lambda i: (i, 0, 0)),
        out_shape=jax.ShapeDtypeStruct((B * G, K, 3), jnp.float32),
    )(rows4.reshape(B * G, K, 8), center.reshape(B * G, 1, 3))

    return neighborhood.reshape(B, G, K, 3), center
